# Initial kernel scaffold; baseline (speedup 1.0000x reference)
import jax
import jax.numpy as jnp
from jax.experimental import pallas as pl


def kernel(x, batch, weight):
    raise NotImplementedError("write your pallas kernel here")


# speedup vs baseline: 3.1349x; 3.1349x over previous
import functools

import jax
import jax.numpy as jnp
from jax import lax
from jax.experimental import pallas as pl
from jax.experimental.pallas import tpu as pltpu
from jax.experimental.pallas import tpu_sc as plsc

N = 10000
D = 512
G = 2048

NC = 2
NS = 16
QW = 128
SUB = 80
NSUB = 8
CHUNK = SUB * NSUB
N_PAD = CHUNK * NS
ROWS_PER_SUB = G // NS


def _sc_segment_sums(x_pad, idx2d, valid):
  mesh = plsc.VectorSubcoreMesh(core_axis_name="c", subcore_axis_name="s")

  @functools.partial(
      pl.kernel,
      out_type=(
          jax.ShapeDtypeStruct((4 * G, QW), jnp.float32),
          jax.ShapeDtypeStruct((NC * G, 16), jnp.float32),
      ),
      mesh=mesh,
      scratch_types=[
          pltpu.VMEM((SUB, QW), jnp.float32),
          pltpu.VMEM((SUB, QW), jnp.float32),
          pltpu.VMEM((SUB, 16), jnp.float32),
          [pltpu.VMEM((SUB,), jnp.int32) for _ in range(NSUB)],
          pltpu.VMEM_SHARED((G, QW), jnp.float32),
          pltpu.VMEM_SHARED((G, QW), jnp.float32),
          pltpu.VMEM_SHARED((G, 16), jnp.float32),
      ],
  )
  def sc_kernel(x_hbm, idx_hbm, valid_hbm, sums_out, counts_out,
                xbuf0, xbuf1, vbuf, idxbuf, acc0, acc1, scnt):
    c = lax.axis_index("c")
    s = lax.axis_index("s")

    def _zrow(i, _):
      def _zcol(j, _):
        xbuf0[i, pl.ds(j * 16, 16)] = jnp.zeros((16,), jnp.float32)
        return 0
      lax.fori_loop(0, QW // 16, _zcol, 0)
      vbuf[i, :] = jnp.zeros((16,), jnp.float32)
      return 0

    lax.fori_loop(0, SUB, _zrow, 0)

    base = s * ROWS_PER_SUB
    rem = ROWS_PER_SUB - SUB
    for acc in (acc0, acc1):
      pltpu.sync_copy(xbuf0, acc.at[pl.ds(base, SUB), :])
      pltpu.sync_copy(xbuf0.at[pl.ds(0, rem), :],
                      acc.at[pl.ds(base + SUB, rem), :])
    pltpu.sync_copy(vbuf, scnt.at[pl.ds(base, SUB), :])
    pltpu.sync_copy(vbuf.at[pl.ds(0, rem), :],
                    scnt.at[pl.ds(base + SUB, rem), :])

    plsc.subcore_barrier()

    for j in range(NSUB):
      pltpu.sync_copy(idx_hbm.at[s * NSUB + j], idxbuf[j])

    for j in range(NSUB):
      r0 = s * CHUNK + j * SUB
      pltpu.sync_copy(x_hbm.at[pl.ds(r0, SUB), pl.ds((2 * c) * QW, QW)],
                      xbuf0)
      pltpu.sync_copy(x_hbm.at[pl.ds(r0, SUB), pl.ds((2 * c + 1) * QW, QW)],
                      xbuf1)
      pltpu.sync_copy(valid_hbm.at[pl.ds(r0, SUB), :], vbuf)
      pltpu.sync_copy(xbuf0, acc0.at[idxbuf[j]], add=True)
      pltpu.sync_copy(xbuf1, acc1.at[idxbuf[j]], add=True)
      pltpu.sync_copy(vbuf, scnt.at[idxbuf[j]], add=True)

    plsc.subcore_barrier()

    pltpu.sync_copy(acc0.at[pl.ds(base, ROWS_PER_SUB), :],
                    sums_out.at[pl.ds((2 * c) * G + base, ROWS_PER_SUB), :])
    pltpu.sync_copy(acc1.at[pl.ds(base, ROWS_PER_SUB), :],
                    sums_out.at[pl.ds((2 * c + 1) * G + base, ROWS_PER_SUB), :])
    pltpu.sync_copy(scnt.at[pl.ds(base, ROWS_PER_SUB), :],
                    counts_out.at[pl.ds(c * G + base, ROWS_PER_SUB), :])

  return sc_kernel(x_pad, idx2d, valid)


def _tc_pool_matmul_body(s0_ref, s1_ref, s2_ref, s3_ref, cnt_ref, w_ref,
                         o_ref):
  sums = jnp.concatenate(
      [s0_ref[...], s1_ref[...], s2_ref[...], s3_ref[...]], axis=1)
  cnt = cnt_ref[...][:, 0:1]
  pooled = sums / jnp.maximum(cnt, 1.0)
  o_ref[...] = jnp.dot(pooled, w_ref[...], preferred_element_type=jnp.float32)


def _tc_pool_matmul(sums, counts, weight):
  blk = 256
  nblk = G // blk
  qspecs = [
      pl.BlockSpec((blk, QW),
                   functools.partial(lambda q, i: (i + q * nblk, 0), q))
      for q in range(4)
  ]
  return pl.pallas_call(
      _tc_pool_matmul_body,
      grid=(nblk,),
      in_specs=qspecs + [
          pl.BlockSpec((blk, 16), lambda i: (i, 0)),
          pl.BlockSpec((D, D), lambda i: (0, 0)),
      ],
      out_specs=pl.BlockSpec((blk, D), lambda i: (i, 0)),
      out_shape=jax.ShapeDtypeStruct((G, D), jnp.float32),
  )(sums, sums, sums, sums, counts, weight)


def kernel(x, batch, weight):
  batch = batch.astype(jnp.int32)
  x_pad = jnp.pad(x, ((0, N_PAD - N), (0, 0)))
  idx2d = jnp.pad(batch, (0, N_PAD - N)).reshape(N_PAD // SUB, SUB)
  valid = jnp.zeros((N_PAD, 16), jnp.float32).at[:N].set(1.0)
  sums, counts = _sc_segment_sums(x_pad, idx2d, valid)
  return _tc_pool_matmul(sums, counts, weight)


# speedup vs baseline: 3.7832x; 1.2068x over previous
import functools

import jax
import jax.numpy as jnp
from jax import lax
from jax.experimental import pallas as pl
from jax.experimental.pallas import tpu as pltpu
from jax.experimental.pallas import tpu_sc as plsc

N = 10000
D = 512
G = 2048

NC = 2
NS = 16
QW = 128
SUB = 80
NCHUNK = N // SUB
NJ = NCHUNK // NS
NREM = NCHUNK - NJ * NS
ROWS_PER_SUB = G // NS


def _sc_segment_sums(x, idx2d, ones_in):
  mesh = plsc.VectorSubcoreMesh(core_axis_name="c", subcore_axis_name="s")

  @functools.partial(
      pl.kernel,
      out_type=(
          jax.ShapeDtypeStruct((4 * G, QW), jnp.float32),
          jax.ShapeDtypeStruct((NC * G, 16), jnp.float32),
      ),
      mesh=mesh,
      scratch_types=[
          pltpu.VMEM((SUB, QW), jnp.float32),
          pltpu.VMEM((SUB, QW), jnp.float32),
          pltpu.VMEM((SUB, 16), jnp.float32),
          [pltpu.VMEM((SUB,), jnp.int32) for _ in range(NJ + 1)],
          pltpu.VMEM_SHARED((G, QW), jnp.float32),
          pltpu.VMEM_SHARED((G, QW), jnp.float32),
          pltpu.VMEM_SHARED((G, 16), jnp.float32),
      ],
  )
  def sc_kernel(x_hbm, idx_hbm, ones_hbm, sums_out, counts_out,
                xbuf0, xbuf1, ones, idxbuf, acc0, acc1, scnt):
    c = lax.axis_index("c")
    s = lax.axis_index("s")

    def _zrow(i, _):
      def _zcol(j, _):
        xbuf0[i, pl.ds(j * 16, 16)] = jnp.zeros((16,), jnp.float32)
        return 0
      lax.fori_loop(0, QW // 16, _zcol, 0)
      ones[i, :] = jnp.zeros((16,), jnp.float32)
      return 0

    lax.fori_loop(0, SUB, _zrow, 0)

    base = s * ROWS_PER_SUB
    rem = ROWS_PER_SUB - SUB
    for acc in (acc0, acc1):
      pltpu.sync_copy(xbuf0, acc.at[pl.ds(base, SUB), :])
      pltpu.sync_copy(xbuf0.at[pl.ds(0, rem), :],
                      acc.at[pl.ds(base + SUB, rem), :])
    pltpu.sync_copy(ones, scnt.at[pl.ds(base, SUB), :])
    pltpu.sync_copy(ones.at[pl.ds(0, rem), :],
                    scnt.at[pl.ds(base + SUB, rem), :])

    plsc.subcore_barrier()

    for j in range(NJ):
      pltpu.sync_copy(idx_hbm.at[j * NS + s], idxbuf[j])

    @pl.when(s < NREM)
    def _():
      pltpu.sync_copy(idx_hbm.at[NJ * NS + s], idxbuf[NJ])

    for j in range(NJ):
      r0 = (j * NS + s) * SUB
      pltpu.sync_copy(x_hbm.at[pl.ds(r0, SUB), pl.ds((2 * c) * QW, QW)],
                      xbuf0)
      pltpu.sync_copy(x_hbm.at[pl.ds(r0, SUB), pl.ds((2 * c + 1) * QW, QW)],
                      xbuf1)
      pltpu.sync_copy(ones_hbm, ones)
      pltpu.sync_copy(xbuf0, acc0.at[idxbuf[j]], add=True)
      pltpu.sync_copy(xbuf1, acc1.at[idxbuf[j]], add=True)
      pltpu.sync_copy(ones, scnt.at[idxbuf[j]], add=True)

    @pl.when(s < NREM)
    def _():
      r0 = (NJ * NS + s) * SUB
      pltpu.sync_copy(x_hbm.at[pl.ds(r0, SUB), pl.ds((2 * c) * QW, QW)],
                      xbuf0)
      pltpu.sync_copy(x_hbm.at[pl.ds(r0, SUB), pl.ds((2 * c + 1) * QW, QW)],
                      xbuf1)
      pltpu.sync_copy(ones_hbm, ones)
      pltpu.sync_copy(xbuf0, acc0.at[idxbuf[NJ]], add=True)
      pltpu.sync_copy(xbuf1, acc1.at[idxbuf[NJ]], add=True)
      pltpu.sync_copy(ones, scnt.at[idxbuf[NJ]], add=True)

    plsc.subcore_barrier()

    pltpu.sync_copy(acc0.at[pl.ds(base, ROWS_PER_SUB), :],
                    sums_out.at[pl.ds((2 * c) * G + base, ROWS_PER_SUB), :])
    pltpu.sync_copy(acc1.at[pl.ds(base, ROWS_PER_SUB), :],
                    sums_out.at[pl.ds((2 * c + 1) * G + base, ROWS_PER_SUB), :])
    pltpu.sync_copy(scnt.at[pl.ds(base, ROWS_PER_SUB), :],
                    counts_out.at[pl.ds(c * G + base, ROWS_PER_SUB), :])

  return sc_kernel(x, idx2d, ones_in)


def _tc_pool_matmul_body(s0_ref, s1_ref, s2_ref, s3_ref, cnt_ref, w_ref,
                         o_ref):
  sums = jnp.concatenate(
      [s0_ref[...], s1_ref[...], s2_ref[...], s3_ref[...]], axis=1)
  cnt = cnt_ref[...][:, 0:1]
  pooled = sums / jnp.maximum(cnt, 1.0)
  o_ref[...] = jnp.dot(pooled, w_ref[...], preferred_element_type=jnp.float32)


def _tc_pool_matmul(sums, counts, weight):
  blk = 256
  nblk = G // blk
  qspecs = [
      pl.BlockSpec((blk, QW),
                   functools.partial(lambda q, i: (i + q * nblk, 0), q))
      for q in range(4)
  ]
  return pl.pallas_call(
      _tc_pool_matmul_body,
      grid=(nblk,),
      in_specs=qspecs + [
          pl.BlockSpec((blk, 16), lambda i: (i, 0)),
          pl.BlockSpec((D, D), lambda i: (0, 0)),
      ],
      out_specs=pl.BlockSpec((blk, D), lambda i: (i, 0)),
      out_shape=jax.ShapeDtypeStruct((G, D), jnp.float32),
  )(sums, sums, sums, sums, counts, weight)


def kernel(x, batch, weight):
  batch = batch.astype(jnp.int32)
  idx2d = batch.reshape(NCHUNK, SUB)
  ones_in = jnp.ones((SUB, 16), jnp.float32)
  sums, counts = _sc_segment_sums(x, idx2d, ones_in)
  return _tc_pool_matmul(sums, counts, weight)
